# baseline (speedup 1.0000x reference)
import jax
import jax.numpy as jnp
import numpy as np
from jax.experimental import pallas as pl
from jax.experimental.pallas import tpu as pltpu

B, S, D = 8, 512, 256
MAX_TOKENS = 512
H, DH, DFF = 8, 32, 1024
K_TOTAL = S * D

ROUTER_CHUNK = 4096
ROUTER_STEPS = K_TOTAL // ROUTER_CHUNK
EPI = ROUTER_STEPS
GRID = ROUTER_STEPS + 1 + B


def _fused_kernel(xf_ref, w1_ref, x3_ref, x_any, mask_ref, b1_ref, w2_ref,
                  b2_ref, wp_ref, wpb_ref,
                  ln1g_ref, ln1b_ref, wqkv_ref, bqkv_ref, wo_ref, bo_ref,
                  ln2g_ref, ln2b_ref, wff1_ref, bff1_ref, wff2_ref, bff2_ref,
                  out_ref,
                  acc_ref, wscr_ref, bias_ref, selw_ref, flags_ref,
                  xbuf_ref, obuf_ref, base_sems, in_sem, out_sem):
    i = pl.program_id(0)

    @pl.when(i == 0)
    def _init():
        acc_ref[...] = jnp.zeros_like(acc_ref)

    @pl.when(i < ROUTER_STEPS)
    def _router():
        acc_ref[...] += jnp.dot(
            xf_ref[...].astype(jnp.bfloat16),
            w1_ref[...].astype(jnp.bfloat16),
            preferred_element_type=jnp.float32,
        )

    @pl.when(i < B)
    def _early():
        wrow = jnp.sum(x3_ref[0] * wp_ref[...], axis=1) + wpb_ref[0, 0]
        wscr_ref[pl.ds(i, 1), :] = wrow[None, :]
        pltpu.make_async_copy(
            x_any.at[pl.ds(i, 1)], out_ref.at[pl.ds(i, 1)], base_sems.at[i]
        ).start()

    @pl.when(i == EPI)
    def _epilogue():
        hdn = acc_ref[...] + b1_ref[...]
        hdn = jnp.where(hdn >= 0, hdn, 0.01 * hdn)
        kl = jnp.sum(hdn * w2_ref[...], axis=1, keepdims=True) + b2_ref[0, 0]
        thr = jnp.clip(
            jax.nn.sigmoid(kl) * MAX_TOKENS, 1.0, float(MAX_TOKENS)
        ).astype(jnp.int32).astype(jnp.float32)
        sel = wscr_ref[...] > thr
        flags_ref[...] = jnp.sum(jnp.where(sel, 1.0, 0.0), axis=1,
                                 keepdims=True)
        bias_ref[...] = mask_ref[:, 0, :] + jnp.where(sel, 0.0, -1e9)
        selw_ref[...] = jnp.where(sel, wscr_ref[...], 0.0)

    @pl.when(i > EPI)
    def _block_step():
        b = i - (EPI + 1)
        pltpu.make_async_copy(
            x_any.at[pl.ds(b, 1)], out_ref.at[pl.ds(b, 1)], base_sems.at[b]
        ).wait()
        any_sel = flags_ref[pl.ds(b, 1), :][0, 0] > 0.0

        @pl.when(any_sel)
        def _block():
            fetch = pltpu.make_async_copy(
                x_any.at[pl.ds(b, 1)], xbuf_ref, in_sem)
            fetch.start()
            fetch.wait()
            xs = xbuf_ref[0]
            bias = bias_ref[pl.ds(b, 1), :][0]
            selw = selw_ref[pl.ds(b, 1), :][0]

            def ln(v, g, bb):
                mu = jnp.mean(v, axis=1, keepdims=True)
                var = jnp.mean((v - mu) ** 2, axis=1, keepdims=True)
                return (v - mu) / jnp.sqrt(var + 1e-5) * g + bb

            def mm(a, w):
                return jax.lax.dot_general(
                    a.astype(jnp.bfloat16), w.astype(jnp.bfloat16),
                    (((1,), (0,)), ((), ())),
                    preferred_element_type=jnp.float32,
                )

            a = ln(xs, ln1g_ref[...], ln1b_ref[...])
            qkv = mm(a, wqkv_ref[...]) + bqkv_ref[...]

            ctx_parts = []
            for h in range(H):
                q = qkv[:, h * DH:(h + 1) * DH]
                k = qkv[:, D + h * DH:D + (h + 1) * DH]
                v = qkv[:, 2 * D + h * DH:2 * D + (h + 1) * DH]
                s = jax.lax.dot_general(
                    q.astype(jnp.bfloat16), k.astype(jnp.bfloat16),
                    (((1,), (1,)), ((), ())),
                    preferred_element_type=jnp.float32,
                ) * (1.0 / np.sqrt(DH)) + bias[None, :]
                m = jnp.max(s, axis=1, keepdims=True)
                p = jnp.exp(s - m)
                p = p / jnp.sum(p, axis=1, keepdims=True)
                ctx_parts.append(mm(p, v))
            ctx = jnp.concatenate(ctx_parts, axis=1)

            h1 = xs + mm(ctx, wo_ref[...]) + bo_ref[...]
            m2 = ln(h1, ln2g_ref[...], ln2b_ref[...])
            ff = jax.nn.gelu(mm(m2, wff1_ref[...]) + bff1_ref[...])
            blk = h1 + mm(ff, wff2_ref[...]) + bff2_ref[...]

            obuf_ref[0] = xs + selw[:, None] * blk
            put = pltpu.make_async_copy(
                obuf_ref, out_ref.at[pl.ds(b, 1)], out_sem)
            put.start()
            put.wait()


def kernel(x, attention_mask, wp_w, wp_b, kp_w1, kp_b1, kp_w2, kp_b2,
           ln1_g, ln1_b, w_qkv, b_qkv, w_o, b_o, ln2_g, ln2_b,
           w_ff1, b_ff1, w_ff2, b_ff2):
    x_flat = x.reshape(B, K_TOTAL)

    def rstep(i):
        return jnp.minimum(i, ROUTER_STEPS - 1)

    row = lambda v: v.reshape(1, -1)
    const = lambda shape: pl.BlockSpec(shape, lambda i: tuple(0 for _ in shape))

    out = pl.pallas_call(
        _fused_kernel,
        grid=(GRID,),
        in_specs=[
            pl.BlockSpec((B, ROUTER_CHUNK), lambda i: (0, rstep(i))),
            pl.BlockSpec((ROUTER_CHUNK, 512), lambda i: (rstep(i), 0)),
            pl.BlockSpec((1, S, D), lambda i: (jnp.minimum(i, B - 1), 0, 0)),
            pl.BlockSpec(memory_space=pl.MemorySpace.ANY),
            const((B, 1, S)),
            const((1, 512)),
            const((1, 512)),
            const((1, 128)),
            const((1, D)),
            const((1, 128)),
            const((1, D)), const((1, D)),
            const((D, 3 * D)), const((1, 3 * D)),
            const((D, D)), const((1, D)),
            const((1, D)), const((1, D)),
            const((D, DFF)), const((1, DFF)),
            const((DFF, D)), const((1, D)),
        ],
        out_specs=pl.BlockSpec(memory_space=pl.MemorySpace.ANY),
        out_shape=jax.ShapeDtypeStruct((B, S, D), jnp.float32),
        scratch_shapes=[
            pltpu.VMEM((B, 512), jnp.float32),
            pltpu.VMEM((B, S), jnp.float32),
            pltpu.VMEM((B, S), jnp.float32),
            pltpu.VMEM((B, S), jnp.float32),
            pltpu.VMEM((B, 1), jnp.float32),
            pltpu.VMEM((1, S, D), jnp.float32),
            pltpu.VMEM((1, S, D), jnp.float32),
            pltpu.SemaphoreType.DMA((B,)),
            pltpu.SemaphoreType.DMA,
            pltpu.SemaphoreType.DMA,
        ],
    )(
        x_flat, kp_w1, x, x,
        attention_mask.reshape(B, 1, S),
        kp_b1.reshape(1, 512), kp_w2.reshape(1, 512),
        jnp.broadcast_to(kp_b2.reshape(1, 1), (1, 128)),
        wp_w.reshape(1, D),
        jnp.broadcast_to(wp_b.reshape(1, 1), (1, 128)),
        row(ln1_g), row(ln1_b),
        w_qkv, row(b_qkv),
        w_o, row(b_o),
        row(ln2_g), row(ln2_b),
        w_ff1, row(b_ff1),
        w_ff2, row(b_ff2),
    )
    return out


# speedup vs baseline: 1.4995x; 1.4995x over previous
import jax
import jax.numpy as jnp
import numpy as np
from jax.experimental import pallas as pl
from jax.experimental.pallas import tpu as pltpu

B, S, D = 8, 512, 256
MAX_TOKENS = 512
H, DH, DFF = 8, 32, 1024
K_TOTAL = S * D

ROUTER_CHUNK = 4096
ROUTER_STEPS = K_TOTAL // ROUTER_CHUNK
EPI = ROUTER_STEPS
GRID = ROUTER_STEPS + 1 + B


def _fused_kernel(xf_ref, w1_ref, x3_ref, x_any, mask_ref, b1_ref, w2_ref,
                  b2_ref, wp_ref, wpb_ref,
                  ln1g_ref, ln1b_ref, wqkv_ref, bqkv_ref, wo_ref, bo_ref,
                  ln2g_ref, ln2b_ref, wff1_ref, bff1_ref, wff2_ref, bff2_ref,
                  out_ref,
                  acc_ref, wscr_ref, bias_ref, selw_ref, flags_ref,
                  xbuf_ref, obuf_ref, base_sems, in_sem, out_sem):
    i = pl.program_id(0)

    @pl.when(i == 0)
    def _init():
        acc_ref[...] = jnp.zeros_like(acc_ref)

    @pl.when(i < ROUTER_STEPS)
    def _router():
        acc_ref[...] += jnp.dot(
            xf_ref[...].astype(jnp.bfloat16),
            w1_ref[...].astype(jnp.bfloat16),
            preferred_element_type=jnp.float32,
        )

    @pl.when(i < B)
    def _early():
        wrow = jnp.sum(x3_ref[0] * wp_ref[...], axis=1) + wpb_ref[0, 0]
        wscr_ref[pl.ds(i, 1), :] = wrow[None, :]

        @pl.when(i >= 1)
        def _prev_stage_done():
            pltpu.make_async_copy(
                xbuf_ref, out_ref.at[pl.ds(i - 1, 1)], base_sems.at[i - 1]
            ).wait()

        xbuf_ref[...] = x3_ref[...]
        pltpu.make_async_copy(
            xbuf_ref, out_ref.at[pl.ds(i, 1)], base_sems.at[i]
        ).start()

    @pl.when(i == EPI)
    def _epilogue():
        hdn = acc_ref[...] + b1_ref[...]
        hdn = jnp.where(hdn >= 0, hdn, 0.01 * hdn)
        kl = jnp.sum(hdn * w2_ref[...], axis=1, keepdims=True) + b2_ref[0, 0]
        thr = jnp.clip(
            jax.nn.sigmoid(kl) * MAX_TOKENS, 1.0, float(MAX_TOKENS)
        ).astype(jnp.int32).astype(jnp.float32)
        sel = wscr_ref[...] > thr
        flags_ref[...] = jnp.sum(jnp.where(sel, 1.0, 0.0), axis=1,
                                 keepdims=True)
        bias_ref[...] = mask_ref[:, 0, :] + jnp.where(sel, 0.0, -1e9)
        selw_ref[...] = jnp.where(sel, wscr_ref[...], 0.0)

    @pl.when(i > EPI)
    def _block_step():
        b = i - (EPI + 1)
        @pl.when(b == 0)
        def _last_stage_done():
            pltpu.make_async_copy(
                xbuf_ref, out_ref.at[pl.ds(B - 1, 1)], base_sems.at[B - 1]
            ).wait()

        any_sel = flags_ref[pl.ds(b, 1), :][0, 0] > 0.0

        @pl.when(any_sel)
        def _block():
            fetch = pltpu.make_async_copy(
                x_any.at[pl.ds(b, 1)], xbuf_ref, in_sem)
            fetch.start()
            fetch.wait()
            xs = xbuf_ref[0]
            bias = bias_ref[pl.ds(b, 1), :][0]
            selw = selw_ref[pl.ds(b, 1), :][0]

            def ln(v, g, bb):
                mu = jnp.mean(v, axis=1, keepdims=True)
                var = jnp.mean((v - mu) ** 2, axis=1, keepdims=True)
                return (v - mu) / jnp.sqrt(var + 1e-5) * g + bb

            def mm(a, w):
                return jax.lax.dot_general(
                    a.astype(jnp.bfloat16), w.astype(jnp.bfloat16),
                    (((1,), (0,)), ((), ())),
                    preferred_element_type=jnp.float32,
                )

            a = ln(xs, ln1g_ref[...], ln1b_ref[...])
            qkv = mm(a, wqkv_ref[...]) + bqkv_ref[...]

            ctx_parts = []
            for h in range(H):
                q = qkv[:, h * DH:(h + 1) * DH]
                k = qkv[:, D + h * DH:D + (h + 1) * DH]
                v = qkv[:, 2 * D + h * DH:2 * D + (h + 1) * DH]
                s = jax.lax.dot_general(
                    q.astype(jnp.bfloat16), k.astype(jnp.bfloat16),
                    (((1,), (1,)), ((), ())),
                    preferred_element_type=jnp.float32,
                ) * (1.0 / np.sqrt(DH)) + bias[None, :]
                m = jnp.max(s, axis=1, keepdims=True)
                p = jnp.exp(s - m)
                p = p / jnp.sum(p, axis=1, keepdims=True)
                ctx_parts.append(mm(p, v))
            ctx = jnp.concatenate(ctx_parts, axis=1)

            h1 = xs + mm(ctx, wo_ref[...]) + bo_ref[...]
            m2 = ln(h1, ln2g_ref[...], ln2b_ref[...])
            ff = jax.nn.gelu(mm(m2, wff1_ref[...]) + bff1_ref[...])
            blk = h1 + mm(ff, wff2_ref[...]) + bff2_ref[...]

            obuf_ref[0] = xs + selw[:, None] * blk
            put = pltpu.make_async_copy(
                obuf_ref, out_ref.at[pl.ds(b, 1)], out_sem)
            put.start()
            put.wait()


def kernel(x, attention_mask, wp_w, wp_b, kp_w1, kp_b1, kp_w2, kp_b2,
           ln1_g, ln1_b, w_qkv, b_qkv, w_o, b_o, ln2_g, ln2_b,
           w_ff1, b_ff1, w_ff2, b_ff2):
    x_flat = x.reshape(B, K_TOTAL)

    def rstep(i):
        return jnp.minimum(i, ROUTER_STEPS - 1)

    row = lambda v: v.reshape(1, -1)
    const = lambda shape: pl.BlockSpec(shape, lambda i: tuple(0 for _ in shape))

    out = pl.pallas_call(
        _fused_kernel,
        grid=(GRID,),
        in_specs=[
            pl.BlockSpec((B, ROUTER_CHUNK), lambda i: (0, rstep(i))),
            pl.BlockSpec((ROUTER_CHUNK, 512), lambda i: (rstep(i), 0)),
            pl.BlockSpec((1, S, D), lambda i: (jnp.minimum(i, B - 1), 0, 0)),
            pl.BlockSpec(memory_space=pl.MemorySpace.ANY),
            const((B, 1, S)),
            const((1, 512)),
            const((1, 512)),
            const((1, 128)),
            const((1, D)),
            const((1, 128)),
            const((1, D)), const((1, D)),
            const((D, 3 * D)), const((1, 3 * D)),
            const((D, D)), const((1, D)),
            const((1, D)), const((1, D)),
            const((D, DFF)), const((1, DFF)),
            const((DFF, D)), const((1, D)),
        ],
        out_specs=pl.BlockSpec(memory_space=pl.MemorySpace.ANY),
        out_shape=jax.ShapeDtypeStruct((B, S, D), jnp.float32),
        scratch_shapes=[
            pltpu.VMEM((B, 512), jnp.float32),
            pltpu.VMEM((B, S), jnp.float32),
            pltpu.VMEM((B, S), jnp.float32),
            pltpu.VMEM((B, S), jnp.float32),
            pltpu.VMEM((B, 1), jnp.float32),
            pltpu.VMEM((1, S, D), jnp.float32),
            pltpu.VMEM((1, S, D), jnp.float32),
            pltpu.SemaphoreType.DMA((B,)),
            pltpu.SemaphoreType.DMA,
            pltpu.SemaphoreType.DMA,
        ],
    )(
        x_flat, kp_w1, x, x,
        attention_mask.reshape(B, 1, S),
        kp_b1.reshape(1, 512), kp_w2.reshape(1, 512),
        jnp.broadcast_to(kp_b2.reshape(1, 1), (1, 128)),
        wp_w.reshape(1, D),
        jnp.broadcast_to(wp_b.reshape(1, 1), (1, 128)),
        row(ln1_g), row(ln1_b),
        w_qkv, row(b_qkv),
        w_o, row(b_o),
        row(ln2_g), row(ln2_b),
        w_ff1, row(b_ff1),
        w_ff2, row(b_ff2),
    )
    return out


# speedup vs baseline: 1.5041x; 1.0031x over previous
import jax
import jax.numpy as jnp
import numpy as np
from jax.experimental import pallas as pl
from jax.experimental.pallas import tpu as pltpu

B, S, D = 8, 512, 256
MAX_TOKENS = 512
H, DH, DFF = 8, 32, 1024
K_TOTAL = S * D

ROUTER_CHUNK = 4096
ROUTER_STEPS = K_TOTAL // ROUTER_CHUNK
EPI = ROUTER_STEPS
GRID = ROUTER_STEPS + 1 + B
EARLY_STRIDE = ROUTER_STEPS // B


def _fused_kernel(xf_ref, w1_ref, x3_ref, x_any, mask_ref, b1_ref, w2_ref,
                  b2_ref, wp_ref, wpb_ref,
                  ln1g_ref, ln1b_ref, wqkv_ref, bqkv_ref, wo_ref, bo_ref,
                  ln2g_ref, ln2b_ref, wff1_ref, bff1_ref, wff2_ref, bff2_ref,
                  out_ref,
                  acc_ref, wscr_ref, bias_ref, selw_ref, flags_ref,
                  xbuf_ref, obuf_ref, base_sems, in_sem, out_sem):
    i = pl.program_id(0)

    @pl.when(i == 0)
    def _init():
        acc_ref[...] = jnp.zeros_like(acc_ref)

    @pl.when(i < ROUTER_STEPS)
    def _router():
        acc_ref[...] += jnp.dot(
            xf_ref[...].astype(jnp.bfloat16),
            w1_ref[...].astype(jnp.bfloat16),
            preferred_element_type=jnp.float32,
        )

    @pl.when(jnp.logical_and(i < B * EARLY_STRIDE, i % EARLY_STRIDE == 0))
    def _early():
        s = i // EARLY_STRIDE
        wrow = jnp.sum(x3_ref[0] * wp_ref[...], axis=1) + wpb_ref[0, 0]
        wscr_ref[pl.ds(s, 1), :] = wrow[None, :]

        @pl.when(s >= 1)
        def _prev_stage_done():
            pltpu.make_async_copy(
                xbuf_ref, out_ref.at[pl.ds(s - 1, 1)], base_sems.at[s - 1]
            ).wait()

        xbuf_ref[...] = x3_ref[...]
        pltpu.make_async_copy(
            xbuf_ref, out_ref.at[pl.ds(s, 1)], base_sems.at[s]
        ).start()

    @pl.when(i == EPI)
    def _epilogue():
        hdn = acc_ref[...] + b1_ref[...]
        hdn = jnp.where(hdn >= 0, hdn, 0.01 * hdn)
        kl = jnp.sum(hdn * w2_ref[...], axis=1, keepdims=True) + b2_ref[0, 0]
        thr = jnp.clip(
            jax.nn.sigmoid(kl) * MAX_TOKENS, 1.0, float(MAX_TOKENS)
        ).astype(jnp.int32).astype(jnp.float32)
        sel = wscr_ref[...] > thr
        flags_ref[...] = jnp.sum(jnp.where(sel, 1.0, 0.0), axis=1,
                                 keepdims=True)
        bias_ref[...] = mask_ref[:, 0, :] + jnp.where(sel, 0.0, -1e9)
        selw_ref[...] = jnp.where(sel, wscr_ref[...], 0.0)

    @pl.when(i > EPI)
    def _block_step():
        b = i - (EPI + 1)
        @pl.when(b == 0)
        def _last_stage_done():
            pltpu.make_async_copy(
                xbuf_ref, out_ref.at[pl.ds(B - 1, 1)], base_sems.at[B - 1]
            ).wait()

        any_sel = flags_ref[pl.ds(b, 1), :][0, 0] > 0.0

        @pl.when(any_sel)
        def _block():
            fetch = pltpu.make_async_copy(
                x_any.at[pl.ds(b, 1)], xbuf_ref, in_sem)
            fetch.start()
            fetch.wait()
            xs = xbuf_ref[0]
            bias = bias_ref[pl.ds(b, 1), :][0]
            selw = selw_ref[pl.ds(b, 1), :][0]

            def ln(v, g, bb):
                mu = jnp.mean(v, axis=1, keepdims=True)
                var = jnp.mean((v - mu) ** 2, axis=1, keepdims=True)
                return (v - mu) / jnp.sqrt(var + 1e-5) * g + bb

            def mm(a, w):
                return jax.lax.dot_general(
                    a.astype(jnp.bfloat16), w.astype(jnp.bfloat16),
                    (((1,), (0,)), ((), ())),
                    preferred_element_type=jnp.float32,
                )

            a = ln(xs, ln1g_ref[...], ln1b_ref[...])
            qkv = mm(a, wqkv_ref[...]) + bqkv_ref[...]

            ctx_parts = []
            for h in range(H):
                q = qkv[:, h * DH:(h + 1) * DH]
                k = qkv[:, D + h * DH:D + (h + 1) * DH]
                v = qkv[:, 2 * D + h * DH:2 * D + (h + 1) * DH]
                s = jax.lax.dot_general(
                    q.astype(jnp.bfloat16), k.astype(jnp.bfloat16),
                    (((1,), (1,)), ((), ())),
                    preferred_element_type=jnp.float32,
                ) * (1.0 / np.sqrt(DH)) + bias[None, :]
                m = jnp.max(s, axis=1, keepdims=True)
                p = jnp.exp(s - m)
                p = p / jnp.sum(p, axis=1, keepdims=True)
                ctx_parts.append(mm(p, v))
            ctx = jnp.concatenate(ctx_parts, axis=1)

            h1 = xs + mm(ctx, wo_ref[...]) + bo_ref[...]
            m2 = ln(h1, ln2g_ref[...], ln2b_ref[...])
            ff = jax.nn.gelu(mm(m2, wff1_ref[...]) + bff1_ref[...])
            blk = h1 + mm(ff, wff2_ref[...]) + bff2_ref[...]

            obuf_ref[0] = xs + selw[:, None] * blk
            put = pltpu.make_async_copy(
                obuf_ref, out_ref.at[pl.ds(b, 1)], out_sem)
            put.start()
            put.wait()


def kernel(x, attention_mask, wp_w, wp_b, kp_w1, kp_b1, kp_w2, kp_b2,
           ln1_g, ln1_b, w_qkv, b_qkv, w_o, b_o, ln2_g, ln2_b,
           w_ff1, b_ff1, w_ff2, b_ff2):
    x_flat = x.reshape(B, K_TOTAL)

    def rstep(i):
        return jnp.minimum(i, ROUTER_STEPS - 1)

    row = lambda v: v.reshape(1, -1)
    const = lambda shape: pl.BlockSpec(shape, lambda i: tuple(0 for _ in shape))

    out = pl.pallas_call(
        _fused_kernel,
        grid=(GRID,),
        in_specs=[
            pl.BlockSpec((B, ROUTER_CHUNK), lambda i: (0, rstep(i))),
            pl.BlockSpec((ROUTER_CHUNK, 512), lambda i: (rstep(i), 0)),
            pl.BlockSpec(
                (1, S, D),
                lambda i: (jnp.minimum(i // EARLY_STRIDE, B - 1), 0, 0)),
            pl.BlockSpec(memory_space=pl.MemorySpace.ANY),
            const((B, 1, S)),
            const((1, 512)),
            const((1, 512)),
            const((1, 128)),
            const((1, D)),
            const((1, 128)),
            const((1, D)), const((1, D)),
            const((D, 3 * D)), const((1, 3 * D)),
            const((D, D)), const((1, D)),
            const((1, D)), const((1, D)),
            const((D, DFF)), const((1, DFF)),
            const((DFF, D)), const((1, D)),
        ],
        out_specs=pl.BlockSpec(memory_space=pl.MemorySpace.ANY),
        out_shape=jax.ShapeDtypeStruct((B, S, D), jnp.float32),
        scratch_shapes=[
            pltpu.VMEM((B, 512), jnp.float32),
            pltpu.VMEM((B, S), jnp.float32),
            pltpu.VMEM((B, S), jnp.float32),
            pltpu.VMEM((B, S), jnp.float32),
            pltpu.VMEM((B, 1), jnp.float32),
            pltpu.VMEM((1, S, D), jnp.float32),
            pltpu.VMEM((1, S, D), jnp.float32),
            pltpu.SemaphoreType.DMA((B,)),
            pltpu.SemaphoreType.DMA,
            pltpu.SemaphoreType.DMA,
        ],
    )(
        x_flat, kp_w1, x, x,
        attention_mask.reshape(B, 1, S),
        kp_b1.reshape(1, 512), kp_w2.reshape(1, 512),
        jnp.broadcast_to(kp_b2.reshape(1, 1), (1, 128)),
        wp_w.reshape(1, D),
        jnp.broadcast_to(wp_b.reshape(1, 1), (1, 128)),
        row(ln1_g), row(ln1_b),
        w_qkv, row(b_qkv),
        w_o, row(b_o),
        row(ln2_g), row(ln2_b),
        w_ff1, row(b_ff1),
        w_ff2, row(b_ff2),
    )
    return out


# speedup vs baseline: 1.5063x; 1.0015x over previous
import jax
import jax.numpy as jnp
import numpy as np
from jax.experimental import pallas as pl
from jax.experimental.pallas import tpu as pltpu

B, S, D = 8, 512, 256
MAX_TOKENS = 512
H, DH, DFF = 8, 32, 1024
K_TOTAL = S * D

ROUTER_CHUNK = 4096
ROUTER_STEPS = K_TOTAL // ROUTER_CHUNK
EPI = ROUTER_STEPS
GRID = ROUTER_STEPS + B
EARLY_STRIDE = ROUTER_STEPS // B


def _fused_kernel(xf_ref, w1_ref, x3_ref, x_any, mask_ref, b1_ref, w2_ref,
                  b2_ref, wp_ref, wpb_ref,
                  ln1g_ref, ln1b_ref, wqkv_ref, bqkv_ref, wo_ref, bo_ref,
                  ln2g_ref, ln2b_ref, wff1_ref, bff1_ref, wff2_ref, bff2_ref,
                  out_ref,
                  acc_ref, wscr_ref, bias_ref, selw_ref, flags_ref,
                  xbuf_ref, obuf_ref, base_sems, in_sem, out_sem):
    i = pl.program_id(0)

    @pl.when(i == 0)
    def _init():
        acc_ref[...] = jnp.zeros_like(acc_ref)

    @pl.when(i < ROUTER_STEPS)
    def _router():
        acc_ref[...] += jnp.dot(
            xf_ref[...].astype(jnp.bfloat16),
            w1_ref[...].astype(jnp.bfloat16),
            preferred_element_type=jnp.float32,
        )

    @pl.when(jnp.logical_and(i < B * EARLY_STRIDE, i % EARLY_STRIDE == 0))
    def _early():
        s = i // EARLY_STRIDE
        wrow = jnp.sum(x3_ref[0] * wp_ref[...], axis=1) + wpb_ref[0, 0]
        wscr_ref[pl.ds(s, 1), :] = wrow[None, :]

        @pl.when(s >= 1)
        def _prev_stage_done():
            pltpu.make_async_copy(
                x3_ref, out_ref.at[pl.ds(s - 1, 1)], base_sems.at[s - 1]
            ).wait()

        pltpu.make_async_copy(
            x3_ref, out_ref.at[pl.ds(s, 1)], base_sems.at[s]
        ).start()

    @pl.when(i == ROUTER_STEPS - 1)
    def _epilogue():
        hdn = acc_ref[...] + b1_ref[...]
        hdn = jnp.where(hdn >= 0, hdn, 0.01 * hdn)
        kl = jnp.sum(hdn * w2_ref[...], axis=1, keepdims=True) + b2_ref[0, 0]
        thr = jnp.clip(
            jax.nn.sigmoid(kl) * MAX_TOKENS, 1.0, float(MAX_TOKENS)
        ).astype(jnp.int32).astype(jnp.float32)
        sel = wscr_ref[...] > thr
        flags_ref[...] = jnp.sum(jnp.where(sel, 1.0, 0.0), axis=1,
                                 keepdims=True)
        bias_ref[...] = mask_ref[:, 0, :] + jnp.where(sel, 0.0, -1e9)
        selw_ref[...] = jnp.where(sel, wscr_ref[...], 0.0)

    @pl.when(i >= ROUTER_STEPS)
    def _block_step():
        b = i - ROUTER_STEPS
        @pl.when(b == 0)
        def _last_stage_done():
            pltpu.make_async_copy(
                xbuf_ref, out_ref.at[pl.ds(B - 1, 1)], base_sems.at[B - 1]
            ).wait()

        any_sel = flags_ref[pl.ds(b, 1), :][0, 0] > 0.0

        @pl.when(any_sel)
        def _block():
            fetch = pltpu.make_async_copy(
                x_any.at[pl.ds(b, 1)], xbuf_ref, in_sem)
            fetch.start()
            fetch.wait()
            xs = xbuf_ref[0]
            bias = bias_ref[pl.ds(b, 1), :][0]
            selw = selw_ref[pl.ds(b, 1), :][0]

            def ln(v, g, bb):
                mu = jnp.mean(v, axis=1, keepdims=True)
                var = jnp.mean((v - mu) ** 2, axis=1, keepdims=True)
                return (v - mu) / jnp.sqrt(var + 1e-5) * g + bb

            def mm(a, w):
                return jax.lax.dot_general(
                    a.astype(jnp.bfloat16), w.astype(jnp.bfloat16),
                    (((1,), (0,)), ((), ())),
                    preferred_element_type=jnp.float32,
                )

            a = ln(xs, ln1g_ref[...], ln1b_ref[...])
            qkv = mm(a, wqkv_ref[...]) + bqkv_ref[...]

            ctx_parts = []
            for h in range(H):
                q = qkv[:, h * DH:(h + 1) * DH]
                k = qkv[:, D + h * DH:D + (h + 1) * DH]
                v = qkv[:, 2 * D + h * DH:2 * D + (h + 1) * DH]
                s = jax.lax.dot_general(
                    q.astype(jnp.bfloat16), k.astype(jnp.bfloat16),
                    (((1,), (1,)), ((), ())),
                    preferred_element_type=jnp.float32,
                ) * (1.0 / np.sqrt(DH)) + bias[None, :]
                m = jnp.max(s, axis=1, keepdims=True)
                p = jnp.exp(s - m)
                p = p / jnp.sum(p, axis=1, keepdims=True)
                ctx_parts.append(mm(p, v))
            ctx = jnp.concatenate(ctx_parts, axis=1)

            h1 = xs + mm(ctx, wo_ref[...]) + bo_ref[...]
            m2 = ln(h1, ln2g_ref[...], ln2b_ref[...])
            ff = jax.nn.gelu(mm(m2, wff1_ref[...]) + bff1_ref[...])
            blk = h1 + mm(ff, wff2_ref[...]) + bff2_ref[...]

            obuf_ref[0] = xs + selw[:, None] * blk
            put = pltpu.make_async_copy(
                obuf_ref, out_ref.at[pl.ds(b, 1)], out_sem)
            put.start()
            put.wait()


def kernel(x, attention_mask, wp_w, wp_b, kp_w1, kp_b1, kp_w2, kp_b2,
           ln1_g, ln1_b, w_qkv, b_qkv, w_o, b_o, ln2_g, ln2_b,
           w_ff1, b_ff1, w_ff2, b_ff2):
    x_flat = x.reshape(B, K_TOTAL)

    def rstep(i):
        return jnp.minimum(i, ROUTER_STEPS - 1)

    row = lambda v: v.reshape(1, -1)
    const = lambda shape: pl.BlockSpec(shape, lambda i: tuple(0 for _ in shape))

    out = pl.pallas_call(
        _fused_kernel,
        grid=(GRID,),
        in_specs=[
            pl.BlockSpec((B, ROUTER_CHUNK), lambda i: (0, rstep(i))),
            pl.BlockSpec((ROUTER_CHUNK, 512), lambda i: (rstep(i), 0)),
            pl.BlockSpec(
                (1, S, D),
                lambda i: (jnp.minimum(i // EARLY_STRIDE, B - 1), 0, 0)),
            pl.BlockSpec(memory_space=pl.MemorySpace.ANY),
            const((B, 1, S)),
            const((1, 512)),
            const((1, 512)),
            const((1, 128)),
            const((1, D)),
            const((1, 128)),
            const((1, D)), const((1, D)),
            const((D, 3 * D)), const((1, 3 * D)),
            const((D, D)), const((1, D)),
            const((1, D)), const((1, D)),
            const((D, DFF)), const((1, DFF)),
            const((DFF, D)), const((1, D)),
        ],
        out_specs=pl.BlockSpec(memory_space=pl.MemorySpace.ANY),
        out_shape=jax.ShapeDtypeStruct((B, S, D), jnp.float32),
        scratch_shapes=[
            pltpu.VMEM((B, 512), jnp.float32),
            pltpu.VMEM((B, S), jnp.float32),
            pltpu.VMEM((B, S), jnp.float32),
            pltpu.VMEM((B, S), jnp.float32),
            pltpu.VMEM((B, 1), jnp.float32),
            pltpu.VMEM((1, S, D), jnp.float32),
            pltpu.VMEM((1, S, D), jnp.float32),
            pltpu.SemaphoreType.DMA((B,)),
            pltpu.SemaphoreType.DMA,
            pltpu.SemaphoreType.DMA,
        ],
    )(
        x_flat, kp_w1, x, x,
        attention_mask.reshape(B, 1, S),
        kp_b1.reshape(1, 512), kp_w2.reshape(1, 512),
        jnp.broadcast_to(kp_b2.reshape(1, 1), (1, 128)),
        wp_w.reshape(1, D),
        jnp.broadcast_to(wp_b.reshape(1, 1), (1, 128)),
        row(ln1_g), row(ln1_b),
        w_qkv, row(b_qkv),
        w_o, row(b_o),
        row(ln2_g), row(ln2_b),
        w_ff1, row(b_ff1),
        w_ff2, row(b_ff2),
    )
    return out


# speedup vs baseline: 1.5181x; 1.0078x over previous
import jax
import jax.numpy as jnp
import numpy as np
from jax.experimental import pallas as pl
from jax.experimental.pallas import tpu as pltpu

B, S, D = 8, 512, 256
MAX_TOKENS = 512
H, DH, DFF = 8, 32, 1024
K_TOTAL = S * D

ROUTER_CHUNK = 4096
TOK = ROUTER_CHUNK // D
ROUTER_STEPS = K_TOTAL // ROUTER_CHUNK
GRID = ROUTER_STEPS + B


def _fused_kernel(xf_ref, w1_ref, bd_ref, x_any, mask_ref, b1_ref, w2_ref,
                  b2_ref, wpb_ref,
                  ln1g_ref, ln1b_ref, wqkv_ref, bqkv_ref, wo_ref, bo_ref,
                  ln2g_ref, ln2b_ref, wff1_ref, bff1_ref, wff2_ref, bff2_ref,
                  out_ref,
                  acc_ref, wscr_ref, thr_ref, flags_ref,
                  xbuf_ref, obuf_ref, stage_ref, base_sems, in_sem, out_sem):
    i = pl.program_id(0)

    @pl.when(i == 0)
    def _init():
        acc_ref[...] = jnp.zeros_like(acc_ref)

    @pl.when(i < ROUTER_STEPS)
    def _router():
        xf_bf = xf_ref[...].astype(jnp.bfloat16)
        acc_ref[...] += jnp.dot(
            xf_bf, w1_ref[...].astype(jnp.bfloat16),
            preferred_element_type=jnp.float32,
        )
        wpart = jax.lax.dot_general(
            xf_bf, bd_ref[...].astype(jnp.bfloat16),
            (((1,), (0,)), ((), ())),
            preferred_element_type=jnp.float32,
        )
        wscr_ref[pl.ds(B * i, B), :] = wpart

        @pl.when(i >= 1)
        def _prev_stage_done():
            pltpu.make_async_copy(
                stage_ref,
                out_ref.at[:, pl.ds(TOK * (i - 1), TOK), :],
                base_sems.at[i - 1],
            ).wait()

        stage_ref[...] = xf_ref[...].reshape(B, TOK, D)
        pltpu.make_async_copy(
            stage_ref,
            out_ref.at[:, pl.ds(TOK * i, TOK), :],
            base_sems.at[i],
        ).start()

    @pl.when(i == ROUTER_STEPS - 1)
    def _epilogue():
        hdn = acc_ref[...] + b1_ref[...]
        hdn = jnp.where(hdn >= 0, hdn, 0.01 * hdn)
        kl = jnp.sum(hdn * w2_ref[...], axis=1, keepdims=True) + b2_ref[0, 0]
        thr = jnp.clip(
            jax.nn.sigmoid(kl) * MAX_TOKENS, 1.0, float(MAX_TOKENS)
        ).astype(jnp.int32).astype(jnp.float32)
        thr_ref[...] = thr
        w3 = wscr_ref[...].reshape(ROUTER_STEPS, B, TOK)
        mxw = jnp.max(jnp.max(w3, axis=0), axis=1, keepdims=True)
        flags_ref[...] = jnp.where(mxw + wpb_ref[0, 0] > thr, 1.0, 0.0)

    @pl.when(i >= ROUTER_STEPS)
    def _block_step():
        b = i - ROUTER_STEPS
        @pl.when(b == 0)
        def _last_stage_done():
            pltpu.make_async_copy(
                stage_ref,
                out_ref.at[:, pl.ds(TOK * (ROUTER_STEPS - 1), TOK), :],
                base_sems.at[ROUTER_STEPS - 1],
            ).wait()

        any_sel = flags_ref[pl.ds(b, 1), :][0, 0] > 0.0

        @pl.when(any_sel)
        def _block():
            fetch = pltpu.make_async_copy(
                x_any.at[pl.ds(b, 1)], xbuf_ref, in_sem)
            fetch.start()
            fetch.wait()
            xs = xbuf_ref[0]
            weights = jnp.concatenate(
                [wscr_ref[pl.ds(B * ii + b, 1), :] for ii in range(ROUTER_STEPS)],
                axis=1,
            )[0] + wpb_ref[0, 0]
            thr_b = thr_ref[pl.ds(b, 1), :][0, 0]
            sel = weights > thr_b
            bias = mask_ref[pl.ds(b, 1)][0, 0] + jnp.where(sel, 0.0, -1e9)
            selw = jnp.where(sel, weights, 0.0)

            def ln(v, g, bb):
                mu = jnp.mean(v, axis=1, keepdims=True)
                var = jnp.mean((v - mu) ** 2, axis=1, keepdims=True)
                return (v - mu) / jnp.sqrt(var + 1e-5) * g + bb

            def mm(a, w):
                return jax.lax.dot_general(
                    a.astype(jnp.bfloat16), w.astype(jnp.bfloat16),
                    (((1,), (0,)), ((), ())),
                    preferred_element_type=jnp.float32,
                )

            a = ln(xs, ln1g_ref[...], ln1b_ref[...])
            qkv = mm(a, wqkv_ref[...]) + bqkv_ref[...]

            ctx_parts = []
            for h in range(H):
                q = qkv[:, h * DH:(h + 1) * DH]
                k = qkv[:, D + h * DH:D + (h + 1) * DH]
                v = qkv[:, 2 * D + h * DH:2 * D + (h + 1) * DH]
                s = jax.lax.dot_general(
                    q.astype(jnp.bfloat16), k.astype(jnp.bfloat16),
                    (((1,), (1,)), ((), ())),
                    preferred_element_type=jnp.float32,
                ) * (1.0 / np.sqrt(DH)) + bias[None, :]
                m = jnp.max(s, axis=1, keepdims=True)
                p = jnp.exp(s - m)
                p = p / jnp.sum(p, axis=1, keepdims=True)
                ctx_parts.append(mm(p, v))
            ctx = jnp.concatenate(ctx_parts, axis=1)

            h1 = xs + mm(ctx, wo_ref[...]) + bo_ref[...]
            m2 = ln(h1, ln2g_ref[...], ln2b_ref[...])
            ff = jax.nn.gelu(mm(m2, wff1_ref[...]) + bff1_ref[...])
            blk = h1 + mm(ff, wff2_ref[...]) + bff2_ref[...]

            obuf_ref[0] = xs + selw[:, None] * blk
            put = pltpu.make_async_copy(
                obuf_ref, out_ref.at[pl.ds(b, 1)], out_sem)
            put.start()
            put.wait()


def kernel(x, attention_mask, wp_w, wp_b, kp_w1, kp_b1, kp_w2, kp_b2,
           ln1_g, ln1_b, w_qkv, b_qkv, w_o, b_o, ln2_g, ln2_b,
           w_ff1, b_ff1, w_ff2, b_ff2):
    x_flat = x.reshape(B, K_TOTAL)
    bd = (jnp.eye(TOK, dtype=jnp.float32)[:, None, :]
          * wp_w[None, :, 0:1]).reshape(ROUTER_CHUNK, TOK)

    def rstep(i):
        return jnp.minimum(i, ROUTER_STEPS - 1)

    row = lambda v: v.reshape(1, -1)
    const = lambda shape: pl.BlockSpec(shape, lambda i: tuple(0 for _ in shape))

    out = pl.pallas_call(
        _fused_kernel,
        grid=(GRID,),
        in_specs=[
            pl.BlockSpec((B, ROUTER_CHUNK), lambda i: (0, rstep(i))),
            pl.BlockSpec((ROUTER_CHUNK, 512), lambda i: (rstep(i), 0)),
            const((ROUTER_CHUNK, TOK)),
            pl.BlockSpec(memory_space=pl.MemorySpace.ANY),
            const((B, 1, S)),
            const((1, 512)),
            const((1, 512)),
            const((1, 128)),
            const((1, 128)),
            const((1, D)), const((1, D)),
            const((D, 3 * D)), const((1, 3 * D)),
            const((D, D)), const((1, D)),
            const((1, D)), const((1, D)),
            const((D, DFF)), const((1, DFF)),
            const((DFF, D)), const((1, D)),
        ],
        out_specs=pl.BlockSpec(memory_space=pl.MemorySpace.ANY),
        out_shape=jax.ShapeDtypeStruct((B, S, D), jnp.float32),
        scratch_shapes=[
            pltpu.VMEM((B, 512), jnp.float32),
            pltpu.VMEM((ROUTER_STEPS * B, TOK), jnp.float32),
            pltpu.VMEM((B, 1), jnp.float32),
            pltpu.VMEM((B, 1), jnp.float32),
            pltpu.VMEM((1, S, D), jnp.float32),
            pltpu.VMEM((1, S, D), jnp.float32),
            pltpu.VMEM((B, TOK, D), jnp.float32),
            pltpu.SemaphoreType.DMA((ROUTER_STEPS,)),
            pltpu.SemaphoreType.DMA,
            pltpu.SemaphoreType.DMA,
        ],
    )(
        x_flat, kp_w1, bd, x,
        attention_mask.reshape(B, 1, S),
        kp_b1.reshape(1, 512), kp_w2.reshape(1, 512),
        jnp.broadcast_to(kp_b2.reshape(1, 1), (1, 128)),
        jnp.broadcast_to(wp_b.reshape(1, 1), (1, 128)),
        row(ln1_g), row(ln1_b),
        w_qkv, row(b_qkv),
        w_o, row(b_o),
        row(ln2_g), row(ln2_b),
        w_ff1, row(b_ff1),
        w_ff2, row(b_ff2),
    )
    return out
